# baseline (speedup 1.0000x reference)
import functools

import jax
import jax.numpy as jnp
from jax import lax
from jax.experimental import pallas as pl
from jax.experimental.pallas import tpu as pltpu
from jax.experimental.pallas import tpu_sc as plsc

D = 768
EPS = 1e-12


_NC = 2
_NS = 16
_NW = _NC * _NS


def _make_sc_gather(c_tok, chunk, c_base):
    b_per_w = c_tok // _NW
    n_chunk = b_per_w // chunk
    mesh = plsc.VectorSubcoreMesh(core_axis_name="c", subcore_axis_name="s")

    @functools.partial(
        pl.kernel,
        mesh=mesh,
        out_type=jax.ShapeDtypeStruct((c_tok, D), jnp.float32),
        scratch_types=[
            pltpu.VMEM((b_per_w,), jnp.int32),
            pltpu.VMEM((chunk, D), jnp.float32),
            pltpu.VMEM((chunk, D), jnp.float32),
            pltpu.SemaphoreType.DMA,
            pltpu.SemaphoreType.DMA,
        ],
    )
    def sc_gather(table_hbm, ids_hbm, out_hbm, idx_v, buf0, buf1, sem0, sem1):
        wid = lax.axis_index("s") * _NC + lax.axis_index("c")
        base = wid * b_per_w
        pltpu.sync_copy(ids_hbm.at[pl.ds(c_base + base, b_per_w)], idx_v)
        bufs = (buf0, buf1)
        sems = (sem0, sem1)

        def start(c):
            return pltpu.async_copy(
                table_hbm.at[idx_v.at[pl.ds(c * chunk, chunk)]],
                bufs[c % 2],
                sems[c % 2],
            )

        handles = [None, None]
        handles[0] = start(0)
        for c in range(n_chunk):
            if c + 1 < n_chunk:
                handles[(c + 1) % 2] = start(c + 1)
            handles[c % 2].wait()
            pltpu.sync_copy(bufs[c % 2], out_hbm.at[pl.ds(base + c * chunk, chunk)])

    return sc_gather


def _ln_chunk(g_chunk, tt_chunk, pos_table, type_table, gamma, beta,
              out_prev, chunk_blk0, n_tok, tok_blk):
    c_tok = g_chunk.shape[0]
    s = pos_table.shape[0]
    n_blk = c_tok // tok_blk
    full = lambda shape: pl.BlockSpec(shape, lambda j: tuple(0 for _ in shape))

    def body(g_ref, tt_ref, pos_ref, type_ref, gamma_ref, beta_ref, *rest):
        o_ref = rest[-1]
        j = pl.program_id(0)
        pos_start = ((chunk_blk0 + j) % (s // tok_blk)) * tok_blk
        x = g_ref[...]
        tt = tt_ref[pl.ds((chunk_blk0 + j) * tok_blk, tok_blk), :]
        ty = type_ref[...]
        seg = ty[0:1, :] + tt * (ty[1:2, :] - ty[0:1, :])
        x = x + pos_ref[pl.ds(pos_start, tok_blk), :] + seg
        mean = jnp.mean(x, axis=-1, keepdims=True)
        xc = x - mean
        var = jnp.mean(xc * xc, axis=-1, keepdims=True)
        y = xc * lax.rsqrt(var + EPS)
        o_ref[...] = y * gamma_ref[...] + beta_ref[...]

    in_specs = [
        pl.BlockSpec((tok_blk, D), lambda j: (j, 0)),
        full((n_tok, 1)),
        full((s, D)),
        full((2, D)),
        full((1, D)),
        full((1, D)),
    ]
    args = [g_chunk, tt_chunk, pos_table, type_table, gamma, beta]
    aliases = {}
    if out_prev is not None:
        in_specs.append(pl.BlockSpec(memory_space=pl.ANY))
        args.append(out_prev)
        aliases = {6: 0}
    return pl.pallas_call(
        body,
        grid=(n_blk,),
        in_specs=in_specs,
        out_specs=pl.BlockSpec((tok_blk, D), lambda j: (chunk_blk0 + j, 0)),
        out_shape=jax.ShapeDtypeStruct((n_tok, D), jnp.float32),
        input_output_aliases=aliases,
    )(*args)


def kernel(input_ids, token_type_ids, word_table, pos_table, type_table, gamma, beta):
    b, s = input_ids.shape
    n_tok = b * s
    ids = input_ids.reshape(n_tok).astype(jnp.int32)
    tt = token_type_ids.reshape(n_tok, 1).astype(jnp.float32)
    tok_blk = 512
    chunks = [(0, 1024, 32), (1024, 3072, 48), (4096, 4096, 64)]
    gathered = [
        _make_sc_gather(size, dma_rows, start)(word_table, ids)
        for start, size, dma_rows in chunks
    ]
    out = None
    for g, (start, size, _) in zip(gathered, chunks):
        out = _ln_chunk(
            g, tt, pos_table, type_table,
            gamma.reshape(1, D), beta.reshape(1, D),
            out, start // tok_blk, n_tok, tok_blk)
    return out.reshape(b, s, D)


# speedup vs baseline: 1.1315x; 1.1315x over previous
import functools

import jax
import jax.numpy as jnp
from jax import lax
from jax.experimental import pallas as pl
from jax.experimental.pallas import tpu as pltpu
from jax.experimental.pallas import tpu_sc as plsc

D = 768
EPS = 1e-12


_NC = 2
_NS = 16
_NW = _NC * _NS


def _make_sc_gather(n_tok, chunk):
    b_per_w = n_tok // _NW
    n_chunk = b_per_w // chunk
    mesh = plsc.VectorSubcoreMesh(core_axis_name="c", subcore_axis_name="s")

    @functools.partial(
        pl.kernel,
        mesh=mesh,
        out_type=jax.ShapeDtypeStruct((n_tok, D), jnp.float32),
        scratch_types=[
            pltpu.VMEM((b_per_w,), jnp.int32),
            pltpu.VMEM((chunk, D), jnp.float32),
            pltpu.VMEM((chunk, D), jnp.float32),
            pltpu.SemaphoreType.DMA,
            pltpu.SemaphoreType.DMA,
        ],
    )
    def sc_gather(table_hbm, ids_hbm, out_hbm, idx_v, buf0, buf1, sem0, sem1):
        wid = lax.axis_index("s") * _NC + lax.axis_index("c")
        base = wid * b_per_w
        pltpu.sync_copy(ids_hbm.at[pl.ds(base, b_per_w)], idx_v)
        bufs = (buf0, buf1)
        sems = (sem0, sem1)

        def start(c):
            return pltpu.async_copy(
                table_hbm.at[idx_v.at[pl.ds(c * chunk, chunk)]],
                bufs[c % 2],
                sems[c % 2],
            )

        handles = [None, None]
        handles[0] = start(0)
        for c in range(n_chunk):
            if c + 1 < n_chunk:
                handles[(c + 1) % 2] = start(c + 1)
            handles[c % 2].wait()
            pltpu.sync_copy(bufs[c % 2], out_hbm.at[pl.ds(base + c * chunk, chunk)])

    return sc_gather


def _ln_chunk(g_chunk, tt_chunk, pos_table, type_table, gamma, beta,
              out_prev, chunk_blk0, n_tok, tok_blk):
    c_tok = g_chunk.shape[0]
    s = pos_table.shape[0]
    n_blk = c_tok // tok_blk
    full = lambda shape: pl.BlockSpec(shape, lambda j: tuple(0 for _ in shape))

    def body(g_ref, tt_ref, pos_ref, type_ref, gamma_ref, beta_ref, *rest):
        o_ref = rest[-1]
        j = pl.program_id(0)
        pos_start = ((chunk_blk0 + j) % (s // tok_blk)) * tok_blk
        x = g_ref[...]
        tt = tt_ref[pl.ds(j * tok_blk, tok_blk), :]
        ty = type_ref[...]
        seg = ty[0:1, :] + tt * (ty[1:2, :] - ty[0:1, :])
        x = x + pos_ref[pl.ds(pos_start, tok_blk), :] + seg
        mean = jnp.mean(x, axis=-1, keepdims=True)
        xc = x - mean
        var = jnp.mean(xc * xc, axis=-1, keepdims=True)
        y = xc * lax.rsqrt(var + EPS)
        o_ref[...] = y * gamma_ref[...] + beta_ref[...]

    in_specs = [
        pl.BlockSpec((tok_blk, D), lambda j: (j, 0)),
        full((c_tok, 1)),
        full((s, D)),
        full((2, D)),
        full((1, D)),
        full((1, D)),
    ]
    args = [g_chunk, tt_chunk, pos_table, type_table, gamma, beta]
    aliases = {}
    if out_prev is not None:
        in_specs.append(pl.BlockSpec(memory_space=pl.ANY))
        args.append(out_prev)
        aliases = {6: 0}
    return pl.pallas_call(
        body,
        grid=(n_blk,),
        in_specs=in_specs,
        out_specs=pl.BlockSpec((tok_blk, D), lambda j: (chunk_blk0 + j, 0)),
        out_shape=jax.ShapeDtypeStruct((n_tok, D), jnp.float32),
        input_output_aliases=aliases,
    )(*args)


def kernel(input_ids, token_type_ids, word_table, pos_table, type_table, gamma, beta):
    b, s = input_ids.shape
    n_tok = b * s
    ids = input_ids.reshape(n_tok).astype(jnp.int32)
    tt = token_type_ids.reshape(n_tok, 1).astype(jnp.float32)
    n_chunks = 2
    tok_blk = 1024
    c_tok = n_tok // n_chunks
    sc_gather = _make_sc_gather(c_tok, 64)
    gathered = [
        sc_gather(word_table, lax.slice(ids, (c * c_tok,), ((c + 1) * c_tok,)))
        for c in range(n_chunks)
    ]
    out = None
    for c in range(n_chunks):
        out = _ln_chunk(
            gathered[c],
            lax.slice(tt, (c * c_tok, 0), ((c + 1) * c_tok, 1)),
            pos_table, type_table,
            gamma.reshape(1, D), beta.reshape(1, D),
            out, c * (c_tok // tok_blk), n_tok, tok_blk)
    return out.reshape(b, s, D)


# speedup vs baseline: 1.1748x; 1.0383x over previous
import functools

import jax
import jax.numpy as jnp
from jax import lax
from jax.experimental import pallas as pl
from jax.experimental.pallas import tpu as pltpu
from jax.experimental.pallas import tpu_sc as plsc

D = 768
EPS = 1e-12


_NC = 2
_NS = 16
_NW = _NC * _NS


def _make_sc_gather(n_tok, chunk):
    b_per_w = n_tok // _NW
    n_chunk = b_per_w // chunk
    mesh = plsc.VectorSubcoreMesh(core_axis_name="c", subcore_axis_name="s")

    @functools.partial(
        pl.kernel,
        mesh=mesh,
        out_type=jax.ShapeDtypeStruct((n_tok, D), jnp.float32),
        scratch_types=[
            pltpu.VMEM((b_per_w,), jnp.int32),
            pltpu.VMEM((chunk, D), jnp.float32),
            pltpu.VMEM((chunk, D), jnp.float32),
            pltpu.SemaphoreType.DMA,
            pltpu.SemaphoreType.DMA,
        ],
    )
    def sc_gather(table_hbm, ids_hbm, out_hbm, idx_v, buf0, buf1, sem0, sem1):
        wid = lax.axis_index("s") * _NC + lax.axis_index("c")
        base = wid * b_per_w
        pltpu.sync_copy(ids_hbm.at[pl.ds(base, b_per_w)], idx_v)
        bufs = (buf0, buf1)
        sems = (sem0, sem1)

        def start(c):
            return pltpu.async_copy(
                table_hbm.at[idx_v.at[pl.ds(c * chunk, chunk)]],
                bufs[c % 2],
                sems[c % 2],
            )

        handles = [None, None]
        handles[0] = start(0)
        for c in range(n_chunk):
            if c + 1 < n_chunk:
                handles[(c + 1) % 2] = start(c + 1)
            handles[c % 2].wait()
            pltpu.sync_copy(bufs[c % 2], out_hbm.at[pl.ds(base + c * chunk, chunk)])

    return sc_gather


def _ln_chunk(g_chunk, tt_chunk, pos_table, type_table, gamma, beta,
              out_prev, chunk_blk0, n_tok, tok_blk):
    c_tok = g_chunk.shape[0]
    s = pos_table.shape[0]
    n_blk = c_tok // tok_blk
    full = lambda shape: pl.BlockSpec(shape, lambda j: tuple(0 for _ in shape))

    def body(g_ref, tt_ref, pos_ref, type_ref, gamma_ref, beta_ref, *rest):
        o_ref = rest[-1]
        j = pl.program_id(0)
        pos_start = ((chunk_blk0 + j) % (s // tok_blk)) * tok_blk
        x = g_ref[...]
        tt = tt_ref[pl.ds(j * tok_blk, tok_blk), :]
        ty = type_ref[...]
        seg = ty[0:1, :] + tt * (ty[1:2, :] - ty[0:1, :])
        x = x + pos_ref[pl.ds(pos_start, tok_blk), :] + seg
        mean = jnp.mean(x, axis=-1, keepdims=True)
        xc = x - mean
        var = jnp.mean(xc * xc, axis=-1, keepdims=True)
        y = xc * lax.rsqrt(var + EPS)
        o_ref[...] = y * gamma_ref[...] + beta_ref[...]

    in_specs = [
        pl.BlockSpec((tok_blk, D), lambda j: (j, 0)),
        full((c_tok, 1)),
        full((s, D)),
        full((2, D)),
        full((1, D)),
        full((1, D)),
    ]
    args = [g_chunk, tt_chunk, pos_table, type_table, gamma, beta]
    aliases = {}
    if out_prev is not None:
        in_specs.append(pl.BlockSpec(memory_space=pl.ANY))
        args.append(out_prev)
        aliases = {6: 0}
    return pl.pallas_call(
        body,
        grid=(n_blk,),
        in_specs=in_specs,
        out_specs=pl.BlockSpec((tok_blk, D), lambda j: (chunk_blk0 + j, 0)),
        out_shape=jax.ShapeDtypeStruct((n_tok, D), jnp.float32),
        input_output_aliases=aliases,
    )(*args)


def kernel(input_ids, token_type_ids, word_table, pos_table, type_table, gamma, beta):
    b, s = input_ids.shape
    n_tok = b * s
    ids = input_ids.reshape(n_tok).astype(jnp.int32)
    tt = token_type_ids.reshape(n_tok, 1).astype(jnp.float32)
    n_chunks = 2
    tok_blk = 2048
    c_tok = n_tok // n_chunks
    sc_gather = _make_sc_gather(c_tok, 64)
    gathered = [
        sc_gather(word_table, lax.slice(ids, (c * c_tok,), ((c + 1) * c_tok,)))
        for c in range(n_chunks)
    ]
    out = None
    for c in range(n_chunks):
        out = _ln_chunk(
            gathered[c],
            lax.slice(tt, (c * c_tok, 0), ((c + 1) * c_tok, 1)),
            pos_table, type_table,
            gamma.reshape(1, D), beta.reshape(1, D),
            out, c * (c_tok // tok_blk), n_tok, tok_blk)
    return out.reshape(b, s, D)
